# baseline (speedup 1.0000x reference)
import functools

import jax
import jax.numpy as jnp
from jax import lax
from jax.experimental import pallas as pl
from jax.experimental.pallas import tpu as pltpu
from jax.experimental.pallas import tpu_sc as plsc

D = 128
V = 10000
C = 10000
NP = 10240
NA = 10112
DUMP = 10000
NC = 2
NS = 16
NW = NC * NS
NQ = 8
QR = NP // NQ
BQ = 128
NCHQ = 6
CAP = NCHQ * BQ
N_ITER = 4


_MM = functools.partial(jnp.dot, preferred_element_type=jnp.float32,
                        precision=lax.Precision.DEFAULT)

_RB = 512


def _mlp2_body(x_ref, w1p, b1p, w2p, b2p, w1n, b1n, w2n, b2n, op_ref, on_ref):
    x = x_ref[...]
    hp = jnp.maximum(_MM(x, w1p[...]) + b1p[...], 0.0)
    op_ref[...] = _MM(hp, w2p[...]) + b2p[...]
    hn = jnp.maximum(_MM(x, w1n[...]) + b1n[...], 0.0)
    on_ref[...] = _MM(hn, w2n[...]) + b2n[...]


def _mlp2(x, pp, pn):
    full = lambda r, c: pl.BlockSpec((r, c), lambda i: (0, 0))
    row = pl.BlockSpec((_RB, D), lambda i: (i, 0))
    return pl.pallas_call(
        _mlp2_body,
        grid=(NP // _RB,),
        in_specs=[row] + [full(D, D), full(1, D)] * 4,
        out_specs=[row, row],
        out_shape=[jax.ShapeDtypeStruct((NP, D), jnp.float32)] * 2,
    )(x, pp["W1"], pp["b1"].reshape(1, D), pp["W2"], pp["b2"].reshape(1, D),
      pn["W1"], pn["b1"].reshape(1, D), pn["W2"], pn["b2"].reshape(1, D))


def _gru_body(p_ref, h_ref, wip, win, bih, whh, bhh, out_ref):
    ap = p_ref[0, 0] + p_ref[1, 0]
    an = p_ref[0, 1] + p_ref[1, 1]
    h = h_ref[...]
    gi = _MM(ap, wip[...]) + _MM(an, win[...]) + bih[...]
    gh = _MM(h, whh[...]) + bhh[...]
    r = jax.nn.sigmoid(gi[:, :D] + gh[:, :D])
    z = jax.nn.sigmoid(gi[:, D:2 * D] + gh[:, D:2 * D])
    n = jnp.tanh(gi[:, 2 * D:] + r * gh[:, 2 * D:])
    out_ref[...] = (1.0 - z) * n + z * h


def _gru(partials, h, p):
    full = lambda r, c: pl.BlockSpec((r, c), lambda i: (0, 0))
    row = pl.BlockSpec((_RB, D), lambda i: (i, 0))
    prow = pl.BlockSpec((NC, 2, _RB, D), lambda i: (0, 0, i, 0))
    wip = p["W_ih"][:, :D].T
    win = p["W_ih"][:, D:].T
    whh = p["W_hh"].T
    return pl.pallas_call(
        _gru_body,
        grid=(NP // _RB,),
        in_specs=[prow, row, full(D, 3 * D), full(D, 3 * D), full(1, 3 * D),
                  full(D, 3 * D), full(1, 3 * D)],
        out_specs=row,
        out_shape=jax.ShapeDtypeStruct((NP, D), jnp.float32),
    )(partials, h, wip, win, p["b_ih"].reshape(1, 3 * D), whh,
      p["b_hh"].reshape(1, 3 * D))


def _make_aggr():
    rpt = NA // NS
    mesh = plsc.VectorSubcoreMesh(core_axis_name="c", subcore_axis_name="s")

    @functools.partial(
        pl.kernel,
        out_type=jax.ShapeDtypeStruct((NC, 2, NP, D), jnp.float32),
        mesh=mesh,
        scratch_types=[
            pltpu.VMEM((NCHQ, BQ), jnp.int32),
            pltpu.VMEM((NCHQ, BQ), jnp.int32),
            pltpu.VMEM((2, BQ, D), jnp.float32),
            pltpu.VMEM_SHARED((NA, D), jnp.float32),
            pltpu.VMEM_SHARED((QR, D), jnp.float32),
            pltpu.SemaphoreType.DMA,
            pltpu.SemaphoreType.DMA,
        ],
    )
    def aggr(mp_hbm, mn_hbm, sp_hbm, dp_hbm, sn_hbm, dn_hbm, zeros_hbm,
             out_hbm, srcv, dstv, rows, accum, tq, gsem, ssem):
        cid = lax.axis_index("c")
        sid = lax.axis_index("s")
        wid = cid * NS + sid
        own = pl.ds(sid * rpt, rpt)
        srows = QR // NS
        for p, (tbl, s_h, d_h) in enumerate(
                ((mp_hbm, sp_hbm, dp_hbm), (mn_hbm, sn_hbm, dn_hbm))):
            pltpu.sync_copy(zeros_hbm, accum.at[own])
            for q in range(NQ):
                pltpu.sync_copy(tbl.at[pl.ds(q * QR + sid * srows, srows)],
                                tq.at[pl.ds(sid * srows, srows)])
                pltpu.sync_copy(s_h.at[wid, q], srcv)
                pltpu.sync_copy(d_h.at[wid, q], dstv)
                plsc.subcore_barrier()

                def fire(j, par):
                    pltpu.async_copy(tq.at[srcv.at[j]], rows.at[par], gsem)

                fire(0, 0)

                def chunk(j, carry):
                    par = lax.rem(j, 2)
                    pltpu.make_async_copy(tq.at[srcv.at[0]],
                                          rows.at[0], gsem).wait()

                    @pl.when(j >= 1)
                    def _():
                        pltpu.make_async_copy(rows.at[0],
                                              accum.at[dstv.at[0]],
                                              ssem).wait()

                    @pl.when(j + 1 < NCHQ)
                    def _():
                        fire(j + 1, 1 - par)

                    pltpu.async_copy(rows.at[par], accum.at[dstv.at[j]],
                                     ssem, add=True)
                    return carry

                lax.fori_loop(0, NCHQ, chunk, 0)
                pltpu.make_async_copy(rows.at[0], accum.at[dstv.at[0]],
                                      ssem).wait()
                plsc.subcore_barrier()
            pltpu.sync_copy(accum.at[own],
                            out_hbm.at[cid, p, pl.ds(sid * rpt, rpt)])

    return aggr


def _bucketize(src, dst):
    key = src // QR
    oh = (key[:, None] == jnp.arange(NQ, dtype=key.dtype)).astype(jnp.int32)
    pos = jnp.take_along_axis(jnp.cumsum(oh, axis=0), key[:, None],
                              axis=1)[:, 0] - 1
    worker = pos % NW
    p2 = pos // NW
    slot = (worker * NQ + key) * CAP + p2
    packed = (src - key * QR) * 16384 + dst
    flat = jnp.full((NW * NQ * CAP,), DUMP, jnp.int32).at[slot].set(packed)
    srcb = (flat // 16384).reshape(NW, NQ, NCHQ, BQ)
    dstb = (flat % 16384).reshape(NW, NQ, NCHQ, BQ)
    return srcb, dstb


def kernel(v_size, c_size, v_edge_index, c_edge_index, p_edge_index,
           n_edge_index, v_emb, c_emb, params):
    aggr = _make_aggr()

    vp = v_edge_index[p_edge_index]
    vn = v_edge_index[n_edge_index]
    cp = c_edge_index[p_edge_index]
    cn = c_edge_index[n_edge_index]
    vp_s, cp_d = _bucketize(vp, cp)
    vn_s, cn_d = _bucketize(vn, cn)
    cp_s, vp_d = _bucketize(cp, vp)
    cn_s, vn_d = _bucketize(cn, vn)
    zeros_tile = jnp.zeros((NA // NS, D), jnp.float32)

    v_h = jnp.pad(v_emb, ((0, NP - V), (0, 0)))
    c_h = jnp.pad(c_emb, ((0, NP - C), (0, 0)))
    v_embs, c_embs = [v_emb], [c_emb]
    for _ in range(N_ITER):
        mvp, mvn = _mlp2(v_h, params["p_v2c"], params["n_v2c"])
        mcp, mcn = _mlp2(c_h, params["p_c2v"], params["n_c2v"])
        part_c = aggr(mvp, mvn, vp_s, cp_d, vn_s, cn_d, zeros_tile)
        part_v = aggr(mcp, mcn, cp_s, vp_d, cn_s, vn_d, zeros_tile)
        c_h = _gru(part_c, c_h, params["c_update"])
        v_h = _gru(part_v, v_h, params["v_update"])
        c_embs.append(c_h[:C])
        v_embs.append(v_h[:V])
    return jnp.stack(v_embs), jnp.stack(c_embs)


# speedup vs baseline: 1.0453x; 1.0453x over previous
import functools

import jax
import jax.numpy as jnp
from jax import lax
from jax.experimental import pallas as pl
from jax.experimental.pallas import tpu as pltpu
from jax.experimental.pallas import tpu_sc as plsc

D = 128
V = 10000
C = 10000
NP = 10240
DUMP = 10000
NC = 2
NS = 16
NW = NC * NS
B = 128
N_ITER = 4


_MM = functools.partial(jnp.dot, preferred_element_type=jnp.float32,
                        precision=lax.Precision.DEFAULT)

_RB = 512


def _mlp2_body(x_ref, w1p, b1p, w2p, b2p, w1n, b1n, w2n, b2n, op_ref, on_ref):
    x = x_ref[...]
    hp = jnp.maximum(_MM(x, w1p[...]) + b1p[...], 0.0)
    op_ref[...] = _MM(hp, w2p[...]) + b2p[...]
    hn = jnp.maximum(_MM(x, w1n[...]) + b1n[...], 0.0)
    on_ref[...] = _MM(hn, w2n[...]) + b2n[...]


def _mlp2(x, pp, pn):
    full = lambda r, c: pl.BlockSpec((r, c), lambda i: (0, 0))
    row = pl.BlockSpec((_RB, D), lambda i: (i, 0))
    return pl.pallas_call(
        _mlp2_body,
        grid=(NP // _RB,),
        in_specs=[row] + [full(D, D), full(1, D)] * 4,
        out_specs=[row, row],
        out_shape=[jax.ShapeDtypeStruct((NP, D), jnp.float32)] * 2,
    )(x, pp["W1"], pp["b1"].reshape(1, D), pp["W2"], pp["b2"].reshape(1, D),
      pn["W1"], pn["b1"].reshape(1, D), pn["W2"], pn["b2"].reshape(1, D))


def _gru_body(p_ref, h_ref, wip, win, bih, whh, bhh, out_ref):
    ap = p_ref[0, 0] + p_ref[1, 0]
    an = p_ref[0, 1] + p_ref[1, 1]
    h = h_ref[...]
    gi = _MM(ap, wip[...]) + _MM(an, win[...]) + bih[...]
    gh = _MM(h, whh[...]) + bhh[...]
    r = jax.nn.sigmoid(gi[:, :D] + gh[:, :D])
    z = jax.nn.sigmoid(gi[:, D:2 * D] + gh[:, D:2 * D])
    n = jnp.tanh(gi[:, 2 * D:] + r * gh[:, 2 * D:])
    out_ref[...] = (1.0 - z) * n + z * h


def _gru(partials, h, p):
    full = lambda r, c: pl.BlockSpec((r, c), lambda i: (0, 0))
    row = pl.BlockSpec((_RB, D), lambda i: (i, 0))
    prow = pl.BlockSpec((NC, 2, _RB, D), lambda i: (0, 0, i, 0))
    wip = p["W_ih"][:, :D].T
    win = p["W_ih"][:, D:].T
    whh = p["W_hh"].T
    return pl.pallas_call(
        _gru_body,
        grid=(NP // _RB,),
        in_specs=[prow, row, full(D, 3 * D), full(D, 3 * D), full(1, 3 * D),
                  full(D, 3 * D), full(1, 3 * D)],
        out_specs=row,
        out_shape=jax.ShapeDtypeStruct((NP, D), jnp.float32),
    )(partials, h, wip, win, p["b_ih"].reshape(1, 3 * D), whh,
      p["b_hh"].reshape(1, 3 * D))


def _make_aggr(nch):
    rpt = NP // NS
    mesh = plsc.VectorSubcoreMesh(core_axis_name="c", subcore_axis_name="s")

    @functools.partial(
        pl.kernel,
        out_type=jax.ShapeDtypeStruct((NC, 2, NP, D), jnp.float32),
        mesh=mesh,
        scratch_types=[
            pltpu.VMEM((nch, B), jnp.int32),
            pltpu.VMEM((nch, B), jnp.int32),
            pltpu.VMEM((2, B, D), jnp.float32),
            pltpu.VMEM_SHARED((NP, D), jnp.float32),
            pltpu.SemaphoreType.DMA,
        ],
    )
    def aggr(mp_hbm, mn_hbm, sp_hbm, dp_hbm, sn_hbm, dn_hbm, zeros_hbm,
             out_hbm, srcv, dstv, rows, accum, sem):
        cid = lax.axis_index("c")
        sid = lax.axis_index("s")
        wid = cid * NS + sid
        own = pl.ds(sid * rpt, rpt)
        for p, (tbl, s_h, d_h) in enumerate(
                ((mp_hbm, sp_hbm, dp_hbm), (mn_hbm, sn_hbm, dn_hbm))):
            pltpu.sync_copy(zeros_hbm, accum.at[own])
            pltpu.sync_copy(s_h.at[wid], srcv)
            pltpu.sync_copy(d_h.at[wid], dstv)
            plsc.subcore_barrier()

            def fire(g, par):
                pltpu.async_copy(tbl.at[srcv.at[g]], rows.at[par], sem)

            fire(0, 0)

            def chunk(g, carry):
                par = lax.rem(g, 2)
                pltpu.make_async_copy(tbl.at[srcv.at[0]],
                                      rows.at[0], sem).wait()

                @pl.when(g + 1 < nch)
                def _():
                    fire(g + 1, 1 - par)

                pltpu.sync_copy(rows.at[par], accum.at[dstv.at[g]], add=True)
                return carry

            lax.fori_loop(0, nch, chunk, 0)
            plsc.subcore_barrier()
            pltpu.sync_copy(accum.at[own], out_hbm.at[cid, p, own])

    return aggr


def _pad_idx(x, ep_pad, nch):
    x = jnp.concatenate(
        [x, jnp.full((ep_pad - x.shape[0],), DUMP, jnp.int32)])
    return x.reshape(NW, nch, B)


def kernel(v_size, c_size, v_edge_index, c_edge_index, p_edge_index,
           n_edge_index, v_emb, c_emb, params):
    ep = p_edge_index.shape[0]
    nch = -(-ep // (NW * B))
    ep_pad = NW * nch * B
    aggr = _make_aggr(nch)

    vp = _pad_idx(v_edge_index[p_edge_index], ep_pad, nch)
    vn = _pad_idx(v_edge_index[n_edge_index], ep_pad, nch)
    cp = _pad_idx(c_edge_index[p_edge_index], ep_pad, nch)
    cn = _pad_idx(c_edge_index[n_edge_index], ep_pad, nch)
    zeros_tile = jnp.zeros((NP // NS, D), jnp.float32)

    v_h = jnp.pad(v_emb, ((0, NP - V), (0, 0)))
    c_h = jnp.pad(c_emb, ((0, NP - C), (0, 0)))
    v_embs, c_embs = [v_emb], [c_emb]
    for _ in range(N_ITER):
        mvp, mvn = _mlp2(v_h, params["p_v2c"], params["n_v2c"])
        mcp, mcn = _mlp2(c_h, params["p_c2v"], params["n_c2v"])
        part_c = aggr(mvp, mvn, vp, cp, vn, cn, zeros_tile)
        part_v = aggr(mcp, mcn, cp, vp, cn, vn, zeros_tile)
        c_h = _gru(part_c, c_h, params["c_update"])
        v_h = _gru(part_v, v_h, params["v_update"])
        c_embs.append(c_h[:C])
        v_embs.append(v_h[:V])
    return jnp.stack(v_embs), jnp.stack(c_embs)
